# baseline (speedup 1.0000x reference)
import functools

import numpy as np

import jax
import jax.numpy as jnp
from jax.experimental import pallas as pl
from jax.experimental.pallas import tpu as pltpu
from jax.experimental.shard_map import shard_map
from jax.sharding import Mesh, PartitionSpec as P

_GAMMA = 0.01
_ALPHA = 0.5
_INF = 1e8


def _rollr1(a):
    return jnp.concatenate([a[:, -1:], a[:, :-1]], axis=1)


def _rolll1(a):
    return jnp.concatenate([a[:, 1:], a[:, :1]], axis=1)


def _dilate_kernel(t_ref, xw_ref, o1_ref, o2_ref, rs_ref, ds_ref, dss_ref,
                   *, L, Bc):
    nd = 2 * L - 1
    ig = 1.0 / _GAMMA
    invl2 = 1.0 / float(L * L)

    t = t_ref[...]
    ii = jax.lax.broadcasted_iota(jnp.int32, (Bc, L), 1)
    tsh = _rolll1(t)

    xw0 = xw_ref[...]
    w0 = xw0[:, :L]
    dd0 = (t - w0) ** 2
    ds_ref[0] = dd0
    dss_ref[1] = (tsh - w0) ** 2
    r0 = jnp.where(ii == 0, dd0, _INF)
    rs_ref[0] = r0

    def fwd(d, carry):
        rp, r2u = carry
        w = pltpu.roll(xw0, d, axis=1)[:, :L]
        dd = (t - w) ** 2
        ds_ref[d] = dd
        dss_ref[d + 1] = (tsh - w) ** 2
        r1u = jnp.where(ii == 0, _INF, _rollr1(rp))
        m = jnp.minimum(jnp.minimum(r2u, r1u), rp)
        s = (jnp.exp((m - r2u) * ig) + jnp.exp((m - r1u) * ig)
             + jnp.exp((m - rp) * ig))
        r = m - _GAMMA * jnp.log(s)
        valid = (ii <= d) & (ii >= d - (L - 1))
        rnew = jnp.where(valid, dd + r, _INF)
        rs_ref[d] = rnew
        return rnew, r1u

    inf_row = jnp.full((Bc, L), _INF, jnp.float32)
    r_last, _ = jax.lax.fori_loop(1, nd, fwd, (r0, inf_row))

    s1 = jnp.sum(r_last[:, L - 1:L])

    e1_0 = jnp.where(ii == L - 1, 1.0, 0.0).astype(jnp.float32)
    zero_row = jnp.zeros((Bc, L), jnp.float32)

    def bwd(k, carry):
        e1, e2s, g1, g2, acc = carry
        d = nd - 2 - k
        rc = rs_ref[d]
        rn1 = rs_ref[d + 1]
        da = dss_ref[d + 1]
        db = ds_ref[d + 1]
        dc = dss_ref[d + 2]
        e1s = jnp.where(ii == L - 1, 0.0, _rolll1(e1))
        wa = jnp.exp(jnp.minimum(g1 - rc - da, 0.0) * ig)
        wb = jnp.exp(jnp.minimum(rn1 - rc - db, 0.0) * ig)
        wc = jnp.exp(jnp.minimum(g2 - rc - dc, 0.0) * ig)
        valid = (ii <= d) & (ii >= d - (L - 1))
        ma = valid & (ii < L - 1)
        mb = valid & (ii >= d - (L - 2))
        mc = ma & (ii >= d - (L - 2))
        enew = (jnp.where(ma, wa * e1s, 0.0)
                + jnp.where(mb, wb * e1, 0.0)
                + jnp.where(mc, wc * e2s, 0.0))
        u = (2 * ii - d).astype(jnp.float32)
        acc = acc + enew * (u * u * invl2)
        gnew = _rolll1(rc)
        return enew, e1s, gnew, g1, acc

    g1_0 = _rolll1(r_last)
    carry0 = (e1_0, zero_row, g1_0, inf_row, zero_row)
    out = jax.lax.fori_loop(0, nd - 1, bwd, carry0)
    acc = out[-1]
    s2 = jnp.sum(acc)

    o1_ref[...] = jnp.full((1, 8, 128), s1, jnp.float32)
    o2_ref[...] = jnp.full((1, 8, 128), s2, jnp.float32)


def _build(L, Bc, nc, interpret=False):
    kern = functools.partial(_dilate_kernel, L=L, Bc=Bc)
    return pl.pallas_call(
        kern,
        grid=(nc,),
        in_specs=[pl.BlockSpec((Bc, L), lambda c: (c, 0)),
                  pl.BlockSpec((Bc, 3 * L), lambda c: (c, 0))],
        out_specs=[pl.BlockSpec((1, 8, 128), lambda c: (c, 0, 0)),
                   pl.BlockSpec((1, 8, 128), lambda c: (c, 0, 0))],
        out_shape=[jax.ShapeDtypeStruct((nc, 8, 128), jnp.float32),
                   jax.ShapeDtypeStruct((nc, 8, 128), jnp.float32)],
        scratch_shapes=[pltpu.VMEM((2 * L - 1, Bc, L), jnp.float32),
                        pltpu.VMEM((2 * L - 1, Bc, L), jnp.float32),
                        pltpu.VMEM((2 * L + 1, Bc, L), jnp.float32)],
        compiler_params=pltpu.CompilerParams(
            dimension_semantics=("arbitrary",)),
        interpret=interpret,
    )


def _chunk_sums(t, xw, L):
    Bl = t.shape[0]
    Bc = 32 if Bl % 32 == 0 else Bl
    nc = Bl // Bc
    o1, o2 = _build(L, Bc, nc)(t, xw)
    return jnp.sum(o1[:, 0, 0]), jnp.sum(o2[:, 0, 0])


@jax.jit
def kernel(input, target):
    B, L, _ = input.shape
    x = input[:, :, 0].astype(jnp.float32)
    t = target[:, :, 0].astype(jnp.float32)
    xpad = jnp.zeros((B, 3 * L), jnp.float32)
    xpad = xpad.at[:, L - 1:2 * L - 1].set(x[:, ::-1])
    xw = jnp.roll(xpad, -(2 * L - 2), axis=1)

    devs = jax.devices()
    ndev = len(devs) if (len(devs) > 1 and B % (32 * len(devs)) == 0) else 1
    if ndev > 1:
        mesh = Mesh(np.array(devs[:ndev]), ("d",))

        def shard_fn(tl, xl):
            a, b = _chunk_sums(tl, xl, L)
            return (jax.lax.psum(a, "d")[None], jax.lax.psum(b, "d")[None])

        s1, s2 = shard_map(shard_fn, mesh=mesh,
                           in_specs=(P("d"), P("d")),
                           out_specs=(P(), P()),
                           check_rep=False)(t, xw)
        s1, s2 = s1[0], s2[0]
    else:
        s1, s2 = _chunk_sums(t, xw, L)
    return _ALPHA * (s1 / B) + (1.0 - _ALPHA) * (s2 / B)


# speedup vs baseline: 2.8265x; 2.8265x over previous
import functools

import jax
import jax.numpy as jnp
from jax.experimental import pallas as pl
from jax.experimental.pallas import tpu as pltpu

_GAMMA = 0.01
_ALPHA = 0.5
_INF = 1e8


def _rollr1(a):
    return jnp.concatenate([a[:, -1:], a[:, :-1]], axis=1)


def _rolll1(a):
    return jnp.concatenate([a[:, 1:], a[:, :1]], axis=1)


def _dilate_kernel(t_ref, xw_ref, o1_ref, o2_ref, rs0_ref, rs1_ref, acc_ref,
                   *, L, Bc, NC):
    nd = 2 * L - 1
    ig = 1.0 / _GAMMA
    invl2 = 1.0 / float(L * L)
    rs_refs = (rs0_ref, rs1_ref)[:NC]

    ii = jax.lax.broadcasted_iota(jnp.int32, (Bc, L), 1)
    ts = [t_ref[pl.ds(c * Bc, Bc), :] for c in range(NC)]
    tshs = [_rolll1(t) for t in ts]
    xws = [xw_ref[pl.ds(c * Bc, Bc), :] for c in range(NC)]

    def window(c, d):
        return pltpu.roll(xws[c], d, axis=1)[:, :L]

    def sm3(a, b, c):
        m = jnp.minimum(jnp.minimum(a, b), c)
        s = (jnp.exp((m - a) * ig) + jnp.exp((m - b) * ig)
             + jnp.exp((m - c) * ig))
        return m - _GAMMA * jnp.log(s)

    r0s = []
    for c in range(NC):
        dd0 = (ts[c] - xws[c][:, :L]) ** 2
        r0 = jnp.where(ii == 0, dd0, _INF)
        rs_refs[c][0] = r0
        r0s.append(r0)

    inf_row = jnp.full((Bc, L), _INF, jnp.float32)

    def fwd(d, carry):
        new = []
        for c in range(NC):
            rp, r2u = carry[2 * c], carry[2 * c + 1]
            dd = (ts[c] - window(c, d)) ** 2
            r1u = jnp.where(ii == 0, _INF, _rollr1(rp))
            valid = (ii <= d) & (ii >= d - (L - 1))
            rnew = jnp.where(valid, dd + sm3(r2u, r1u, rp), _INF)
            rs_refs[c][d] = rnew
            new += [rnew, r1u]
        return tuple(new)

    carry = tuple(v for r0 in r0s for v in (r0, inf_row))
    carry = jax.lax.fori_loop(1, nd, fwd, carry)
    r_lasts = [carry[2 * c] for c in range(NC)]

    s1 = sum(jnp.sum(rl[:, L - 1:L]) for rl in r_lasts)

    e1_0 = jnp.where(ii == L - 1, 1.0, 0.0).astype(jnp.float32)
    zero_row = jnp.zeros((Bc, L), jnp.float32)
    for c in range(NC):
        acc_ref[c] = zero_row

    def bwd(k, carry):
        d = nd - 2 - k
        new = []
        for c in range(NC):
            e1, e2s, w1 = carry[3 * c], carry[3 * c + 1], carry[3 * c + 2]
            rc = rs_refs[c][d]
            rn1 = rs_refs[c][d + 1]
            rn2 = rs_refs[c][d + 2]
            g1 = _rolll1(rn1)
            g2 = _rolll1(rn2)
            w0 = window(c, d)
            da = (tshs[c] - w0) ** 2
            db = (ts[c] - w1) ** 2
            dc = (tshs[c] - w1) ** 2
            e1s = jnp.where(ii == L - 1, 0.0, _rolll1(e1))
            wa = jnp.exp(jnp.minimum(g1 - rc - da, 0.0) * ig)
            wb = jnp.exp(jnp.minimum(rn1 - rc - db, 0.0) * ig)
            wc = jnp.exp(jnp.minimum(g2 - rc - dc, 0.0) * ig)
            valid = (ii <= d) & (ii >= d - (L - 1))
            ma = valid & (ii < L - 1)
            mb = valid & (ii >= d - (L - 2))
            mc = ma & (ii >= d - (L - 2))
            enew = (jnp.where(ma, wa * e1s, 0.0)
                    + jnp.where(mb, wb * e1, 0.0)
                    + jnp.where(mc, wc * e2s, 0.0))
            u = (2 * ii - d).astype(jnp.float32)
            acc_ref[c] = acc_ref[c] + enew * (u * u * invl2)
            new += [enew, e1s, w0]
        return tuple(new)

    for c in range(NC):
        rs_refs[c][nd] = inf_row
    w1_inits = [window(c, nd - 1) for c in range(NC)]
    carry = tuple(v for c in range(NC)
                  for v in (e1_0, zero_row, w1_inits[c]))
    jax.lax.fori_loop(0, nd - 1, bwd, carry)

    s2 = sum(jnp.sum(acc_ref[c]) for c in range(NC))

    o1_ref[...] = jnp.full((1, 8, 128), s1, jnp.float32)
    o2_ref[...] = jnp.full((1, 8, 128), s2, jnp.float32)


def _build(L, B, Bc, interpret=False):
    NC = B // Bc
    kern = functools.partial(_dilate_kernel, L=L, Bc=Bc, NC=NC)
    return pl.pallas_call(
        kern,
        grid=(1,),
        in_specs=[pl.BlockSpec((B, L), lambda c: (0, 0)),
                  pl.BlockSpec((B, 3 * L), lambda c: (0, 0))],
        out_specs=[pl.BlockSpec((1, 8, 128), lambda c: (0, 0, 0)),
                   pl.BlockSpec((1, 8, 128), lambda c: (0, 0, 0))],
        out_shape=[jax.ShapeDtypeStruct((1, 8, 128), jnp.float32),
                   jax.ShapeDtypeStruct((1, 8, 128), jnp.float32)],
        scratch_shapes=[pltpu.VMEM((2 * L, Bc, L), jnp.float32),
                        pltpu.VMEM((2 * L, Bc, L), jnp.float32),
                        pltpu.VMEM((2, Bc, L), jnp.float32)],
        compiler_params=pltpu.CompilerParams(
            dimension_semantics=("arbitrary",)),
        interpret=interpret,
    )


@jax.jit
def kernel(input, target):
    B, L, _ = input.shape
    x = input[:, :, 0].astype(jnp.float32)
    t = target[:, :, 0].astype(jnp.float32)
    Bc = 32 if B % 32 == 0 and B // 32 <= 2 else B
    xpad = jnp.zeros((B, 3 * L), jnp.float32)
    xpad = xpad.at[:, L - 1:2 * L - 1].set(x[:, ::-1])
    xw = jnp.roll(xpad, -(2 * L - 2), axis=1)
    o1, o2 = _build(L, B, Bc)(t, xw)
    return _ALPHA * (o1[0, 0, 0] / B) + (1.0 - _ALPHA) * (o2[0, 0, 0] / B)


# speedup vs baseline: 3.2708x; 1.1572x over previous
import functools

import jax
import jax.numpy as jnp
from jax.experimental import pallas as pl
from jax.experimental.pallas import tpu as pltpu

_GAMMA = 0.01
_ALPHA = 0.5
_INF = 1e8


def _rollr1(a):
    return jnp.concatenate([a[:, -1:], a[:, :-1]], axis=1)


def _rolll1(a):
    return jnp.concatenate([a[:, 1:], a[:, :1]], axis=1)


def _dilate_kernel(t_ref, xw_ref, o1_ref, o2_ref, rs0_ref, rs1_ref, acc_ref,
                   *, L, Bc, NC):
    nd = 2 * L - 1
    ig = 1.0 / _GAMMA
    invl2 = 1.0 / float(L * L)
    rs_refs = (rs0_ref, rs1_ref)[:NC]

    ii = jax.lax.broadcasted_iota(jnp.int32, (Bc, L), 1)
    ts = [t_ref[pl.ds(c * Bc, Bc), :] for c in range(NC)]
    tshs = [_rolll1(t) for t in ts]
    xws = [xw_ref[pl.ds(c * Bc, Bc), :] for c in range(NC)]

    def window(c, d):
        return pltpu.roll(xws[c], d, axis=1)[:, :L]

    def sm3(a, b, c):
        m = jnp.minimum(jnp.minimum(a, b), c)
        s = (jnp.exp((m - a) * ig) + jnp.exp((m - b) * ig)
             + jnp.exp((m - c) * ig))
        return m - _GAMMA * jnp.log(s)

    r0s = []
    for c in range(NC):
        dd0 = (ts[c] - xws[c][:, :L]) ** 2
        r0 = jnp.where(ii == 0, dd0, _INF)
        rs_refs[c][0] = r0
        r0s.append(r0)

    inf_row = jnp.full((Bc, L), _INF, jnp.float32)

    def fwd_step(d, carry):
        new = []
        for c in range(NC):
            rp, r2u = carry[2 * c], carry[2 * c + 1]
            dd = (ts[c] - window(c, d)) ** 2
            r1u = jnp.where(ii == 0, _INF, _rollr1(rp))
            valid = (ii <= d) & (ii >= d - (L - 1))
            rnew = jnp.where(valid, dd + sm3(r2u, r1u, rp), _INF)
            rs_refs[c][d] = rnew
            new += [rnew, r1u]
        return tuple(new)

    def fwd(k, carry):
        return fwd_step(2 * k + 2, fwd_step(2 * k + 1, carry))

    carry = tuple(v for r0 in r0s for v in (r0, inf_row))
    carry = jax.lax.fori_loop(0, (nd - 1) // 2, fwd, carry)
    r_lasts = [carry[2 * c] for c in range(NC)]

    s1 = sum(jnp.sum(rl[:, L - 1:L]) for rl in r_lasts)

    e1_0 = jnp.where(ii == L - 1, 1.0, 0.0).astype(jnp.float32)
    zero_row = jnp.zeros((Bc, L), jnp.float32)
    for c in range(NC):
        acc_ref[c] = zero_row

    def bwd_step(d, carry):
        new = []
        for c in range(NC):
            e1, e2s, w1 = carry[3 * c], carry[3 * c + 1], carry[3 * c + 2]
            rc = rs_refs[c][d]
            rn1 = rs_refs[c][d + 1]
            rn2 = rs_refs[c][d + 2]
            g1 = _rolll1(rn1)
            g2 = _rolll1(rn2)
            w0 = window(c, d)
            da = (tshs[c] - w0) ** 2
            db = (ts[c] - w1) ** 2
            dc = (tshs[c] - w1) ** 2
            e1s = jnp.where(ii == L - 1, 0.0, _rolll1(e1))
            wa = jnp.exp(jnp.minimum(g1 - rc - da, 0.0) * ig)
            wb = jnp.exp(jnp.minimum(rn1 - rc - db, 0.0) * ig)
            wc = jnp.exp(jnp.minimum(g2 - rc - dc, 0.0) * ig)
            valid = (ii <= d) & (ii >= d - (L - 1))
            ma = valid & (ii < L - 1)
            mb = valid & (ii >= d - (L - 2))
            mc = ma & (ii >= d - (L - 2))
            enew = (jnp.where(ma, wa * e1s, 0.0)
                    + jnp.where(mb, wb * e1, 0.0)
                    + jnp.where(mc, wc * e2s, 0.0))
            u = (2 * ii - d).astype(jnp.float32)
            acc_ref[c] = acc_ref[c] + enew * (u * u * invl2)
            new += [enew, e1s, w0]
        return tuple(new)

    def bwd(k, carry):
        return bwd_step(nd - 3 - 2 * k, bwd_step(nd - 2 - 2 * k, carry))

    for c in range(NC):
        rs_refs[c][nd] = inf_row
    w1_inits = [window(c, nd - 1) for c in range(NC)]
    carry = tuple(v for c in range(NC)
                  for v in (e1_0, zero_row, w1_inits[c]))
    jax.lax.fori_loop(0, (nd - 1) // 2, bwd, carry)

    s2 = sum(jnp.sum(acc_ref[c]) for c in range(NC))

    o1_ref[...] = jnp.full((1, 8, 128), s1, jnp.float32)
    o2_ref[...] = jnp.full((1, 8, 128), s2, jnp.float32)


def _build(L, B, Bc, interpret=False):
    NC = B // Bc
    kern = functools.partial(_dilate_kernel, L=L, Bc=Bc, NC=NC)
    return pl.pallas_call(
        kern,
        grid=(1,),
        in_specs=[pl.BlockSpec((B, L), lambda c: (0, 0)),
                  pl.BlockSpec((B, 3 * L), lambda c: (0, 0))],
        out_specs=[pl.BlockSpec((1, 8, 128), lambda c: (0, 0, 0)),
                   pl.BlockSpec((1, 8, 128), lambda c: (0, 0, 0))],
        out_shape=[jax.ShapeDtypeStruct((1, 8, 128), jnp.float32),
                   jax.ShapeDtypeStruct((1, 8, 128), jnp.float32)],
        scratch_shapes=[pltpu.VMEM((2 * L, Bc, L), jnp.float32),
                        pltpu.VMEM((2 * L, Bc, L), jnp.float32),
                        pltpu.VMEM((2, Bc, L), jnp.float32)],
        compiler_params=pltpu.CompilerParams(
            dimension_semantics=("arbitrary",)),
        interpret=interpret,
    )


@jax.jit
def kernel(input, target):
    B, L, _ = input.shape
    x = input[:, :, 0].astype(jnp.float32)
    t = target[:, :, 0].astype(jnp.float32)
    Bc = 32 if B % 32 == 0 and B // 32 <= 2 else B
    xpad = jnp.zeros((B, 3 * L), jnp.float32)
    xpad = xpad.at[:, L - 1:2 * L - 1].set(x[:, ::-1])
    xw = jnp.roll(xpad, -(2 * L - 2), axis=1)
    o1, o2 = _build(L, B, Bc)(t, xw)
    return _ALPHA * (o1[0, 0, 0] / B) + (1.0 - _ALPHA) * (o2[0, 0, 0] / B)


# speedup vs baseline: 3.5377x; 1.0816x over previous
import functools

import jax
import jax.numpy as jnp
from jax.experimental import pallas as pl
from jax.experimental.pallas import tpu as pltpu

_GAMMA = 0.01
_ALPHA = 0.5
_INF = 1e8


def _rollr1(a):
    return jnp.concatenate([a[:, -1:], a[:, :-1]], axis=1)


def _rolll1(a):
    return jnp.concatenate([a[:, 1:], a[:, :1]], axis=1)


def _dilate_kernel(t_ref, xw_ref, o1_ref, o2_ref, rs0_ref, rs1_ref, acc_ref,
                   *, L, Bc, NC):
    nd = 2 * L - 1
    ig = 1.0 / _GAMMA
    invl2 = 1.0 / float(L * L)
    rs_refs = (rs0_ref, rs1_ref)[:NC]

    ii = jax.lax.broadcasted_iota(jnp.int32, (Bc, L), 1)
    ts = [t_ref[pl.ds(c * Bc, Bc), :] for c in range(NC)]
    tshs = [_rolll1(t) for t in ts]
    xws = [xw_ref[pl.ds(c * Bc, Bc), :] for c in range(NC)]

    def window(c, d):
        return pltpu.roll(xws[c], d, axis=1)[:, :L]

    def sm3(a, b, c):
        m = jnp.minimum(jnp.minimum(a, b), c)
        s = (jnp.exp((m - a) * ig) + jnp.exp((m - b) * ig)
             + jnp.exp((m - c) * ig))
        return m - _GAMMA * jnp.log(s)

    r0s = []
    for c in range(NC):
        dd0 = (ts[c] - xws[c][:, :L]) ** 2
        r0 = jnp.where(ii == 0, dd0, _INF)
        rs_refs[c][0] = r0
        r0s.append(r0)

    inf_row = jnp.full((Bc, L), _INF, jnp.float32)

    def fwd_step(d, carry):
        new = []
        for c in range(NC):
            rp, r2u = carry[2 * c], carry[2 * c + 1]
            dd = (ts[c] - window(c, d)) ** 2
            r1u = jnp.where(ii == 0, _INF, _rollr1(rp))
            valid = (ii <= d) & (ii >= d - (L - 1))
            rnew = jnp.where(valid, dd + sm3(r2u, r1u, rp), _INF)
            rs_refs[c][d] = rnew
            new += [rnew, r1u]
        return tuple(new)

    U = 3 if (nd - 1) % 3 == 0 else 2

    def fwd(k, carry):
        for u in range(U):
            carry = fwd_step(U * k + 1 + u, carry)
        return carry

    carry = tuple(v for r0 in r0s for v in (r0, inf_row))
    carry = jax.lax.fori_loop(0, (nd - 1) // U, fwd, carry)
    r_lasts = [carry[2 * c] for c in range(NC)]

    s1 = sum(jnp.sum(rl[:, L - 1:L]) for rl in r_lasts)

    e1_0 = jnp.where(ii == L - 1, 1.0, 0.0).astype(jnp.float32)
    zero_row = jnp.zeros((Bc, L), jnp.float32)
    for c in range(NC):
        acc_ref[c] = zero_row

    def bwd_step(d, carry):
        new = []
        for c in range(NC):
            e1, e2s, w1 = carry[3 * c], carry[3 * c + 1], carry[3 * c + 2]
            rc = rs_refs[c][d]
            rn1 = rs_refs[c][d + 1]
            rn2 = rs_refs[c][d + 2]
            g1 = _rolll1(rn1)
            g2 = _rolll1(rn2)
            w0 = window(c, d)
            da = (tshs[c] - w0) ** 2
            db = (ts[c] - w1) ** 2
            dc = (tshs[c] - w1) ** 2
            e1s = jnp.where(ii == L - 1, 0.0, _rolll1(e1))
            wa = jnp.exp(jnp.minimum(g1 - rc - da, 0.0) * ig)
            wb = jnp.exp(jnp.minimum(rn1 - rc - db, 0.0) * ig)
            wc = jnp.exp(jnp.minimum(g2 - rc - dc, 0.0) * ig)
            valid = (ii <= d) & (ii >= d - (L - 1))
            ma = valid & (ii < L - 1)
            mb = valid & (ii >= d - (L - 2))
            mc = ma & (ii >= d - (L - 2))
            enew = (jnp.where(ma, wa * e1s, 0.0)
                    + jnp.where(mb, wb * e1, 0.0)
                    + jnp.where(mc, wc * e2s, 0.0))
            u = (2 * ii - d).astype(jnp.float32)
            acc_ref[c] = acc_ref[c] + enew * (u * u * invl2)
            new += [enew, e1s, w0]
        return tuple(new)

    def bwd(k, carry):
        for u in range(U):
            carry = bwd_step(nd - 2 - U * k - u, carry)
        return carry

    for c in range(NC):
        rs_refs[c][nd] = inf_row
    w1_inits = [window(c, nd - 1) for c in range(NC)]
    carry = tuple(v for c in range(NC)
                  for v in (e1_0, zero_row, w1_inits[c]))
    jax.lax.fori_loop(0, (nd - 1) // U, bwd, carry)

    s2 = sum(jnp.sum(acc_ref[c]) for c in range(NC))

    o1_ref[...] = jnp.full((1, 8, 128), s1, jnp.float32)
    o2_ref[...] = jnp.full((1, 8, 128), s2, jnp.float32)


def _build(L, B, Bc, interpret=False):
    NC = B // Bc
    kern = functools.partial(_dilate_kernel, L=L, Bc=Bc, NC=NC)
    return pl.pallas_call(
        kern,
        grid=(1,),
        in_specs=[pl.BlockSpec((B, L), lambda c: (0, 0)),
                  pl.BlockSpec((B, 3 * L), lambda c: (0, 0))],
        out_specs=[pl.BlockSpec((1, 8, 128), lambda c: (0, 0, 0)),
                   pl.BlockSpec((1, 8, 128), lambda c: (0, 0, 0))],
        out_shape=[jax.ShapeDtypeStruct((1, 8, 128), jnp.float32),
                   jax.ShapeDtypeStruct((1, 8, 128), jnp.float32)],
        scratch_shapes=[pltpu.VMEM((2 * L, Bc, L), jnp.float32),
                        pltpu.VMEM((2 * L, Bc, L), jnp.float32),
                        pltpu.VMEM((2, Bc, L), jnp.float32)],
        compiler_params=pltpu.CompilerParams(
            dimension_semantics=("arbitrary",)),
        interpret=interpret,
    )


@jax.jit
def kernel(input, target):
    B, L, _ = input.shape
    x = input[:, :, 0].astype(jnp.float32)
    t = target[:, :, 0].astype(jnp.float32)
    Bc = 32 if B % 32 == 0 and B // 32 <= 2 else B
    xpad = jnp.zeros((B, 3 * L), jnp.float32)
    xpad = xpad.at[:, L - 1:2 * L - 1].set(x[:, ::-1])
    xw = jnp.roll(xpad, -(2 * L - 2), axis=1)
    o1, o2 = _build(L, B, Bc)(t, xw)
    return _ALPHA * (o1[0, 0, 0] / B) + (1.0 - _ALPHA) * (o2[0, 0, 0] / B)
